# baseline (speedup 1.0000x reference)
import functools

import jax
import jax.numpy as jnp
from jax import lax
from jax.experimental import pallas as pl
from jax.experimental.pallas import tpu as pltpu
from jax.experimental.pallas import tpu_sc as plsc

D = 256
DD = 512
MV_CHUNK = 2048
MV_STEPS = 8


def _ohf(x, n):
    i = lax.broadcasted_iota(jnp.int32, (x.shape[0], n), 1)
    return (i == x).astype(jnp.float32)


def _bitsf(x, nbits):
    i = lax.broadcasted_iota(jnp.int32, (x.shape[0], nbits), 1)
    return (jnp.bitwise_and(x, jnp.left_shift(jnp.int32(1), i)) != 0
            ).astype(jnp.float32)


def _fuse_body(st_ref, ab_ref, it_ref, mv_ref,
               whp_ref, wlv_ref, wac_ref, woh_ref,
               wsp_ref, wab_ref, wit_ref, wmv_ref,
               bhp_ref, blv_ref, bac_ref, boh_ref,
               bsp_ref, bab_ref, bit_ref, bmv_ref,
               wg_ref, wv_ref,
               t_hp, t_lv, t_gsbtn, t_tsfa, t_sp, t_ab, t_itfx, t_mvpp, t_r,
               mv_f, pp_blk):
    step = pl.program_id(0)

    def dot(a, b):
        return jnp.dot(a, b, preferred_element_type=jnp.float32,
                       precision=lax.Precision.HIGHEST)

    def mcat(i):
        return jnp.concatenate([wg_ref[i], wv_ref[i]], axis=1)

    @pl.when(step == 0)
    def _():
        m0 = mcat(0)
        m4 = mcat(4)
        b0 = (bhp_ref[...] + blv_ref[...] + bac_ref[...] + boh_ref[...])
        bias0 = dot(b0, m0)

        r10 = lax.broadcasted_iota(jnp.int32, (1024, 1), 0)
        t_hp[...] = dot(_bitsf(r10, 10), dot(whp_ref[...], m0)) + bias0
        r7 = lax.broadcasted_iota(jnp.int32, (128, 1), 0)
        t_lv[...] = dot(_bitsf(r7, 7), dot(wlv_ref[...], m0))

        m31 = dot(woh_ref[...], m0)
        r = lax.broadcasted_iota(jnp.int32, (864, 1), 0)
        e1 = jnp.concatenate([
            jnp.zeros((864, 1), jnp.float32),
            _ohf(r // 216, 3), _ohf((r // 27) % 8, 7),
            _ohf((r // 9) % 3, 2), _ohf((r // 3) % 3, 2), _ohf(r % 3, 2),
            jnp.zeros((864, 14), jnp.float32)], axis=1)
        t_gsbtn[...] = dot(e1, m31)

        r = lax.broadcasted_iota(jnp.int32, (405, 1), 0)
        e2 = jnp.concatenate([
            jnp.zeros((405, 17), jnp.float32),
            _ohf(r // 45, 8), _ohf((r // 9) % 5, 4), _ohf((r // 3) % 3, 2),
        ], axis=1)
        t_tsfa[...] = dot(e2, m31) + dot(_ohf(r % 3, 2), dot(wac_ref[...], m0))

        m1 = mcat(1)
        t_sp[...] = dot(st_ref[0:256, :], dot(wsp_ref[...], m1)) + dot(bsp_ref[...], m1)
        m2 = mcat(2)
        t_ab[...] = dot(ab_ref[...], dot(wab_ref[...], m2)) + dot(bab_ref[...], m2)

        m3 = mcat(3)
        item_part = (dot(it_ref[...], dot(wit_ref[0:128, :], m3))
                     + dot(bit_ref[...], m3))
        fxm = dot(wit_ref[128:144, :], m3)
        t_itfx[...] = jnp.concatenate(
            [item_part + fxm[f:f + 1, :] for f in range(16)] + [item_part],
            axis=0)

        mv_f[...] = dot(mv_ref[0:256, :], dot(wmv_ref[0:256, :], m4)) + dot(bmv_ref[...], m4)
        r64 = lax.broadcasted_iota(jnp.int32, (64, 1), 0)
        pp_blk[...] = dot(_bitsf(r64, 6), dot(wmv_ref[256:262, :], m4))

        t_r[...] = jnp.concatenate(
            [dot(woh_ref[0:1, :], m0), jnp.zeros((7, DD), jnp.float32)], axis=0)

    mvf = mv_f[...]
    t_mvpp[...] = jnp.concatenate(
        [mvf + pp_blk[pl.ds(8 * step + j, 1), :] for j in range(8)], axis=0)


def _fuse_tables(p):
    f32 = jnp.float32
    shapes = [(1024, DD), (128, DD), (864, DD), (405, DD), (256, DD),
              (256, DD), (4352, DD), (MV_CHUNK * MV_STEPS, DD), (8, DD)]
    outs = [jax.ShapeDtypeStruct(s, f32) for s in shapes]
    nil = lambda i: (0, 0)
    out_specs = [pl.BlockSpec(s, nil) for s in shapes[:-2]] + [
        pl.BlockSpec((MV_CHUNK, DD), lambda i: (i, 0)),
        pl.BlockSpec((8, DD), nil)]
    return pl.pallas_call(
        _fuse_body,
        grid=(MV_STEPS,),
        in_specs=[
            pl.BlockSpec(a.shape, functools.partial(lambda n, i: (0,) * n,
                                                    len(a.shape)))
            for a in _fuse_args(p)],
        out_specs=out_specs,
        out_shape=tuple(outs),
        scratch_shapes=[pltpu.VMEM((256, DD), f32), pltpu.VMEM((64, DD), f32)],
    )(*_fuse_args(p))


def _fuse_args(p):
    return (p['species_table'], p['ability_table'], p['item_table'],
            p['move_table'],
            p['W_hp'], p['W_level'], p['W_active'], p['W_onehot'],
            p['W_species'], p['W_ability'], p['W_item'], p['W_moves'],
            p['b_hp'][None, :], p['b_level'][None, :], p['b_active'][None, :],
            p['b_onehot'][None, :],
            p['b_species'][None, :], p['b_ability'][None, :],
            p['b_item'][None, :], p['b_moves'][None, :],
            p['Wg'], p['Wv'])


S = 16
NW = 32


def _sc_body(act_hbm, side_hbm,
             t_hp, t_lv, t_gsbtn, t_tsfa, t_sp, t_ab, t_itfx, t_mvpp, t_r,
             out_a, out_s,
             e_v, d_hp, d_lv, d_gsb, d_tsf, d_sp, d_ab, d_itfx, d_mv,
             r_v, rat_v, out_v, sem):
    wid = lax.axis_index("s") * 2 + lax.axis_index("c")
    pltpu.sync_copy(t_r.at[pl.ds(0, DD)], r_v)
    lanes = lax.iota(jnp.int32, S)
    rg = [r_v[pl.ds(s * 16, 16)] for s in range(16)]
    rv = [r_v[pl.ds(256 + s * 16, 16)] for s in range(16)]

    def do_chunk(e_hbm, out_hbm, base):
        pltpu.sync_copy(e_hbm.at[pl.ds(base * 24, S * 24)], e_v)

        def feat(c):
            return plsc.load_gather(e_v, [lanes * 24 + c])

        hp = feat(0).astype(jnp.float32)
        mx = jnp.maximum(feat(1).astype(jnp.float32), 1.0)
        ratio = jnp.clip(hp / mx, 0.0, 1.0)
        rat_v[...] = ratio
        i_hp = (1023.0 * ratio).astype(jnp.int32)
        i_lv = jnp.bitwise_and(feat(10), 127)
        g3 = jnp.minimum(feat(2), 3)
        s7 = jnp.minimum(feat(3), 7)
        b2 = jnp.minimum(feat(4), 2)
        t2 = jnp.minimum(feat(5), 2)
        n2 = jnp.minimum(feat(6), 2)
        i_gsb = (((g3 * 8 + s7) * 3 + b2) * 3 + t2) * 3 + n2
        tox = jnp.minimum(feat(7), 8)
        slp = jnp.minimum(feat(8), 4)
        fnt = jnp.minimum(feat(9), 2)
        act = jnp.minimum(feat(11), 2)
        i_tsf = ((tox * 5 + slp) * 3 + fnt) * 3 + act
        i_sp = feat(12)
        i_ab = feat(13)
        i_it = jnp.minimum(feat(15), 16) * 256 + feat(14)
        i_mv = [jnp.bitwise_and(feat(20 + j), 63) * 256 + feat(16 + j)
                for j in range(4)]

        hs = [pltpu.async_copy(t_hp.at[i_hp], d_hp, sem),
              pltpu.async_copy(t_lv.at[i_lv], d_lv, sem),
              pltpu.async_copy(t_gsbtn.at[i_gsb], d_gsb, sem),
              pltpu.async_copy(t_tsfa.at[i_tsf], d_tsf, sem),
              pltpu.async_copy(t_sp.at[i_sp], d_sp, sem),
              pltpu.async_copy(t_ab.at[i_ab], d_ab, sem),
              pltpu.async_copy(t_itfx.at[i_it], d_itfx, sem)]
        hs += [pltpu.async_copy(t_mvpp.at[i_mv[j]],
                                d_mv.at[pl.ds(j * S, S)], sem)
               for j in range(4)]
        for h in hs:
            h.wait()

        def ent(i, carry):
            ri = plsc.load_gather(rat_v, [jnp.broadcast_to(i, (S,))])
            for s in range(16):
                cg = s * 16
                cv = cg + 256
                g0 = (d_hp[i, pl.ds(cg, 16)]
                      + d_lv[i, pl.ds(cg, 16)]
                      + d_gsb[i, pl.ds(cg, 16)]
                      + d_tsf[i, pl.ds(cg, 16)]
                      + ri * rg[s])
                v0 = (d_hp[i, pl.ds(cv, 16)]
                      + d_lv[i, pl.ds(cv, 16)]
                      + d_gsb[i, pl.ds(cv, 16)]
                      + d_tsf[i, pl.ds(cv, 16)]
                      + ri * rv[s])
                g1 = d_sp[i, pl.ds(cg, 16)]
                v1 = d_sp[i, pl.ds(cv, 16)]
                g2 = d_ab[i, pl.ds(cg, 16)]
                v2 = d_ab[i, pl.ds(cv, 16)]
                g3_ = d_itfx[i, pl.ds(cg, 16)]
                v3 = d_itfx[i, pl.ds(cv, 16)]
                g4 = (d_mv[i, pl.ds(cg, 16)]
                      + d_mv[i + S, pl.ds(cg, 16)]
                      + d_mv[i + 2 * S, pl.ds(cg, 16)]
                      + d_mv[i + 3 * S, pl.ds(cg, 16)])
                v4 = (d_mv[i, pl.ds(cv, 16)]
                      + d_mv[i + S, pl.ds(cv, 16)]
                      + d_mv[i + 2 * S, pl.ds(cv, 16)]
                      + d_mv[i + 3 * S, pl.ds(cv, 16)])
                m = jnp.maximum(jnp.maximum(jnp.maximum(g0, g1),
                                            jnp.maximum(g2, g3_)), g4)
                e0 = jnp.exp(g0 - m)
                e1 = jnp.exp(g1 - m)
                e2 = jnp.exp(g2 - m)
                e3 = jnp.exp(g3_ - m)
                e4 = jnp.exp(g4 - m)
                den = e0 + e1 + e2 + e3 + e4
                num = e0 * v0 + e1 * v1 + e2 * v2 + e3 * v3 + e4 * v4
                out_v[i, pl.ds(cg, 16)] = num / den
            return carry

        lax.fori_loop(0, S, ent, 0)
        pltpu.sync_copy(out_v, out_hbm.at[pl.ds(base, S)])

    na = act_hbm.shape[0] // 24 // NW
    ns = side_hbm.shape[0] // 24 // NW

    def loop_a(k, c):
        do_chunk(act_hbm, out_a, wid * na + k * S)
        return c

    def loop_s(k, c):
        do_chunk(side_hbm, out_s, wid * ns + k * S)
        return c

    lax.fori_loop(0, na // S, loop_a, 0)
    lax.fori_loop(0, ns // S, loop_s, 0)


def kernel(active_entities, side_entities, params):
    B = active_entities.shape[0]
    act = active_entities.reshape(-1, 24)
    side = side_entities.reshape(-1, 24)
    NA, NS = act.shape[0], side.shape[0]
    tables = _fuse_tables(params)

    f32 = jnp.float32
    mesh = plsc.VectorSubcoreMesh(core_axis_name="c", subcore_axis_name="s",
                                  num_cores=2, num_subcores=16)
    sc = functools.partial(
        pl.kernel,
        out_type=[jax.ShapeDtypeStruct((NA, D), f32),
                  jax.ShapeDtypeStruct((NS, D), f32)],
        mesh=mesh,
        compiler_params=pltpu.CompilerParams(needs_layout_passes=False),
        scratch_types=[pltpu.VMEM((S * 24,), jnp.int32)]
        + [pltpu.VMEM((S, DD), f32)] * 7
        + [pltpu.VMEM((4 * S, DD), f32),
           pltpu.VMEM((DD,), f32),
           pltpu.VMEM((S,), f32),
           pltpu.VMEM((S, D), f32),
           pltpu.SemaphoreType.DMA],
    )(_sc_body)
    out_a, out_s = sc(act.reshape(-1), side.reshape(-1),
                      *tables[:-1], tables[-1].reshape(-1))

    active_embeddings = out_a.reshape(B, -1, D)
    side_embeddings = out_s.reshape(B, -1, D)
    tok = side_entities[..., 12]
    valid_team_mask = (tok != 0) | (tok != 1)
    return active_embeddings, side_embeddings, valid_team_mask
